# baseline (speedup 1.0000x reference)
import functools

import jax
import jax.numpy as jnp
from jax.experimental import pallas as pl

D_MODEL = 512
HIDDEN = 256
NUM_CLUSTERS = 1024
EPS = 1e-6
U_MIN = 1e-4

_PREC = jax.lax.Precision.DEFAULT


def _fused_kernel(x_ref, u_ref, m_ref, c_ref, wp_ref, bp_ref, wo_ref, bo_ref,
                  out_ref):
    x = x_ref[...]

    h = jax.lax.dot_general(x, wp_ref[...], (((1,), (0,)), ((), ())),
                            preferred_element_type=jnp.float32,
                            precision=_PREC)
    h = jax.nn.gelu(h + bp_ref[...])
    ab = jax.lax.dot_general(h, wo_ref[...], (((1,), (0,)), ((), ())),
                             preferred_element_type=jnp.float32,
                             precision=_PREC)
    ab = ab + bo_ref[...]
    alpha = jnp.clip(jax.nn.softplus(ab[:, 0:1]) + 1.0, 1.0, 10.0)
    beta = jnp.clip(jax.nn.softplus(ab[:, 1:2]) + 1.0, 1.0, 10.0)

    uc = jnp.clip(u_ref[...], U_MIN, 1.0 - U_MIN)
    t = jnp.exp(jnp.log1p(-uc) / (beta + EPS))
    one_minus_t = jnp.clip(1.0 - t, EPS, 1.0)
    g = jnp.exp(jnp.log(one_minus_t) / (alpha + EPS))
    gates = jnp.clip(g, EPS, 1.0 - EPS) * m_ref[...]

    ct = c_ref[...]
    s = jax.lax.dot_general(x, ct, (((1,), (0,)), ((), ())),
                            preferred_element_type=jnp.float32,
                            precision=_PREC)
    c2 = jnp.sum(ct * ct, axis=0, keepdims=True)
    d = c2 - 2.0 * s
    dmin = jnp.min(d, axis=1, keepdims=True)
    entity = (d[:, 0:1] <= dmin).astype(jnp.float32) * m_ref[...]

    out_ref[...] = gates * entity


@functools.partial(jax.jit, static_argnames=())
def kernel(embeddings, attention_mask, centroids, u, W_proj, b_proj, W_out,
           b_out):
    B, L, D = embeddings.shape
    N = B * L
    R = 512
    flat = embeddings.reshape(N, D)
    u2 = u.reshape(N, 1)
    m2 = attention_mask.reshape(N, 1)

    grid = (N // R,)
    out = pl.pallas_call(
        _fused_kernel,
        grid=grid,
        in_specs=[
            pl.BlockSpec((R, D), lambda i: (i, 0)),
            pl.BlockSpec((R, 1), lambda i: (i, 0)),
            pl.BlockSpec((R, 1), lambda i: (i, 0)),
            pl.BlockSpec((D, NUM_CLUSTERS), lambda i: (0, 0)),
            pl.BlockSpec((D, HIDDEN), lambda i: (0, 0)),
            pl.BlockSpec((1, HIDDEN), lambda i: (0, 0)),
            pl.BlockSpec((HIDDEN, 2), lambda i: (0, 0)),
            pl.BlockSpec((1, 2), lambda i: (0, 0)),
        ],
        out_specs=pl.BlockSpec((R, 1), lambda i: (i, 0)),
        out_shape=jax.ShapeDtypeStruct((N, 1), jnp.float32),
    )(flat, u2, m2, centroids.T, W_proj, b_proj.reshape(1, HIDDEN),
      W_out, b_out.reshape(1, 2))
    return out.reshape(B, L)


# speedup vs baseline: 1.1022x; 1.1022x over previous
import functools

import jax
import jax.numpy as jnp
from jax.experimental import pallas as pl

D_MODEL = 512
HIDDEN = 256
NUM_CLUSTERS = 1024
EPS = 1e-6
U_MIN = 1e-4

_PREC = jax.lax.Precision.DEFAULT


def _fused_kernel(x_ref, u_ref, m_ref, c_ref, wp_ref, bp_ref, wo_ref, bo_ref,
                  out_ref):
    x = x_ref[...]

    h = jax.lax.dot_general(x, wp_ref[...], (((1,), (0,)), ((), ())),
                            preferred_element_type=jnp.float32,
                            precision=_PREC)
    h = jax.nn.gelu(h + bp_ref[...])
    ab = jax.lax.dot_general(h, wo_ref[...], (((1,), (0,)), ((), ())),
                             preferred_element_type=jnp.float32,
                             precision=_PREC)
    ab = ab + bo_ref[...]
    alpha = jnp.clip(jax.nn.softplus(ab[:, 0:1]) + 1.0, 1.0, 10.0)
    beta = jnp.clip(jax.nn.softplus(ab[:, 1:2]) + 1.0, 1.0, 10.0)

    ucol = jnp.transpose(u_ref[0])
    uc = jnp.clip(ucol, U_MIN, 1.0 - U_MIN)
    t = jnp.exp(jnp.log1p(-uc) / (beta + EPS))
    one_minus_t = jnp.clip(1.0 - t, EPS, 1.0)
    g = jnp.exp(jnp.log(one_minus_t) / (alpha + EPS))
    gates = jnp.clip(g, EPS, 1.0 - EPS)

    ct = c_ref[...]
    s = jax.lax.dot_general(x, ct, (((1,), (0,)), ((), ())),
                            preferred_element_type=jnp.float32,
                            precision=_PREC)
    c2 = jnp.sum(ct * ct, axis=0, keepdims=True)
    d = c2 - 2.0 * s
    dmin = jnp.min(d, axis=1, keepdims=True)
    entity = (d[:, 0:1] <= dmin).astype(jnp.float32)

    res = jnp.transpose(gates * entity)
    mrow = m_ref[0]
    out_ref[0] = res * mrow * mrow


@functools.partial(jax.jit, static_argnames=())
def kernel(embeddings, attention_mask, centroids, u, W_proj, b_proj, W_out,
           b_out):
    B, L, D = embeddings.shape
    N = B * L
    R = 512
    G = N // R
    flat = embeddings.reshape(N, D)
    u2 = u.reshape(G, 1, R)
    m2 = attention_mask.reshape(G, 1, R)

    out = pl.pallas_call(
        _fused_kernel,
        grid=(G,),
        in_specs=[
            pl.BlockSpec((R, D), lambda i: (i, 0)),
            pl.BlockSpec((1, 1, R), lambda i: (i, 0, 0)),
            pl.BlockSpec((1, 1, R), lambda i: (i, 0, 0)),
            pl.BlockSpec((D, NUM_CLUSTERS), lambda i: (0, 0)),
            pl.BlockSpec((D, HIDDEN), lambda i: (0, 0)),
            pl.BlockSpec((1, HIDDEN), lambda i: (0, 0)),
            pl.BlockSpec((HIDDEN, 2), lambda i: (0, 0)),
            pl.BlockSpec((1, 2), lambda i: (0, 0)),
        ],
        out_specs=pl.BlockSpec((1, 1, R), lambda i: (i, 0, 0)),
        out_shape=jax.ShapeDtypeStruct((G, 1, R), jnp.float32),
    )(flat, u2, m2, centroids.T, W_proj, b_proj.reshape(1, HIDDEN),
      W_out, b_out.reshape(1, 2))
    return out.reshape(B, L)


# speedup vs baseline: 1.2231x; 1.1096x over previous
import functools

import jax
import jax.numpy as jnp
from jax.experimental import pallas as pl
from jax.experimental.pallas import tpu as pltpu

D_MODEL = 512
HIDDEN = 256
NUM_CLUSTERS = 1024
EPS = 1e-6
U_MIN = 1e-4

_PREC = jax.lax.Precision.DEFAULT


def _fused_kernel(x_ref, u_ref, m_ref, c_ref, wp_ref, bp_ref, wo_ref, bo_ref,
                  out_ref, ct_ref):
    @pl.when(pl.program_id(0) == 0)
    def _():
        ct_ref[...] = jnp.transpose(c_ref[...])

    x = x_ref[...]

    h = jax.lax.dot_general(x, wp_ref[...], (((1,), (0,)), ((), ())),
                            preferred_element_type=jnp.float32,
                            precision=_PREC)
    h = jax.nn.gelu(h + bp_ref[...])
    ab = jax.lax.dot_general(h, wo_ref[...], (((1,), (0,)), ((), ())),
                             preferred_element_type=jnp.float32,
                             precision=_PREC)
    ab = ab + bo_ref[...]
    alpha = jnp.clip(jax.nn.softplus(ab[:, 0:1]) + 1.0, 1.0, 10.0)
    beta = jnp.clip(jax.nn.softplus(ab[:, 1:2]) + 1.0, 1.0, 10.0)

    ucol = jnp.transpose(u_ref[0])
    uc = jnp.clip(ucol, U_MIN, 1.0 - U_MIN)
    t = jnp.exp(jnp.log1p(-uc) / (beta + EPS))
    one_minus_t = jnp.clip(1.0 - t, EPS, 1.0)
    g = jnp.exp(jnp.log(one_minus_t) / (alpha + EPS))
    gates = jnp.clip(g, EPS, 1.0 - EPS)

    ct = ct_ref[...]
    s = jax.lax.dot_general(x, ct, (((1,), (0,)), ((), ())),
                            preferred_element_type=jnp.float32,
                            precision=_PREC)
    c2 = jnp.sum(ct * ct, axis=0, keepdims=True)
    d = c2 - 2.0 * s
    dmin = jnp.min(d, axis=1, keepdims=True)
    entity = (d[:, 0:1] <= dmin).astype(jnp.float32)

    res = jnp.transpose(gates * entity)
    mrow = m_ref[0]
    out_ref[0] = res * mrow * mrow


@functools.partial(jax.jit, static_argnames=())
def kernel(embeddings, attention_mask, centroids, u, W_proj, b_proj, W_out,
           b_out):
    B, L, D = embeddings.shape
    N = B * L
    R = 512
    G = N // R
    flat = embeddings.reshape(N, D)
    u2 = u.reshape(G, 1, R)
    m2 = attention_mask.reshape(G, 1, R)

    out = pl.pallas_call(
        _fused_kernel,
        grid=(G,),
        in_specs=[
            pl.BlockSpec((R, D), lambda i: (i, 0)),
            pl.BlockSpec((1, 1, R), lambda i: (i, 0, 0)),
            pl.BlockSpec((1, 1, R), lambda i: (i, 0, 0)),
            pl.BlockSpec((NUM_CLUSTERS, D), lambda i: (0, 0)),
            pl.BlockSpec((D, HIDDEN), lambda i: (0, 0)),
            pl.BlockSpec((1, HIDDEN), lambda i: (0, 0)),
            pl.BlockSpec((HIDDEN, 2), lambda i: (0, 0)),
            pl.BlockSpec((1, 2), lambda i: (0, 0)),
        ],
        out_specs=pl.BlockSpec((1, 1, R), lambda i: (i, 0, 0)),
        out_shape=jax.ShapeDtypeStruct((G, 1, R), jnp.float32),
        scratch_shapes=[pltpu.VMEM((D, NUM_CLUSTERS), jnp.float32)],
    )(flat, u2, m2, centroids, W_proj, b_proj.reshape(1, HIDDEN),
      W_out, b_out.reshape(1, 2))
    return out.reshape(B, L)


# speedup vs baseline: 1.6751x; 1.3696x over previous
import functools

import jax
import jax.numpy as jnp
from jax.experimental import pallas as pl
from jax.experimental.pallas import tpu as pltpu

D_MODEL = 512
HIDDEN = 256
NUM_CLUSTERS = 1024
EPS = 1e-6
U_MIN = 1e-4

_PREC = jax.lax.Precision.DEFAULT


def _fused_kernel(x_ref, u_ref, m_ref, c_ref, wp_ref, bp_ref, wo_ref, bo_ref,
                  out_ref, ct_ref):
    i = pl.program_id(0)

    @pl.when(i == 0)
    def _():
        ct_ref[...] = jnp.transpose(c_ref[...])

    x = x_ref[...]

    h = jax.lax.dot_general(x, wp_ref[...], (((1,), (0,)), ((), ())),
                            preferred_element_type=jnp.float32,
                            precision=_PREC)
    h = jax.nn.gelu(h + bp_ref[...])
    ab = jax.lax.dot_general(h, wo_ref[...], (((1,), (0,)), ((), ())),
                             preferred_element_type=jnp.float32,
                             precision=_PREC)
    ab = ab + bo_ref[...]
    alpha = jnp.clip(jax.nn.softplus(ab[:, 0:1]) + 1.0, 1.0, 10.0)
    beta = jnp.clip(jax.nn.softplus(ab[:, 1:2]) + 1.0, 1.0, 10.0)

    ucol = jnp.transpose(u_ref[pl.ds(i, 1), :])
    uc = jnp.clip(ucol, U_MIN, 1.0 - U_MIN)
    t = jnp.exp(jnp.log1p(-uc) / (beta + EPS))
    one_minus_t = jnp.clip(1.0 - t, EPS, 1.0)
    g = jnp.exp(jnp.log(one_minus_t) / (alpha + EPS))
    gates = jnp.clip(g, EPS, 1.0 - EPS)

    ct = ct_ref[...]
    s = jax.lax.dot_general(x, ct, (((1,), (0,)), ((), ())),
                            preferred_element_type=jnp.float32,
                            precision=_PREC)
    c2 = jnp.sum(ct * ct, axis=0, keepdims=True)
    d = c2 - 2.0 * s
    dmin = jnp.min(d, axis=1, keepdims=True)
    entity = (d[:, 0:1] <= dmin).astype(jnp.float32)

    res = jnp.transpose(gates * entity)
    mrow = m_ref[pl.ds(i, 1), :]
    out_ref[pl.ds(i, 1), :] = res * mrow * mrow


@functools.partial(jax.jit, static_argnames=())
def kernel(embeddings, attention_mask, centroids, u, W_proj, b_proj, W_out,
           b_out):
    B, L, D = embeddings.shape
    N = B * L
    R = L
    flat = embeddings.reshape(N, D)

    out = pl.pallas_call(
        _fused_kernel,
        grid=(B,),
        in_specs=[
            pl.BlockSpec((R, D), lambda i: (i, 0)),
            pl.BlockSpec((B, L), lambda i: (0, 0)),
            pl.BlockSpec((B, L), lambda i: (0, 0)),
            pl.BlockSpec((NUM_CLUSTERS, D), lambda i: (0, 0)),
            pl.BlockSpec((D, HIDDEN), lambda i: (0, 0)),
            pl.BlockSpec((1, HIDDEN), lambda i: (0, 0)),
            pl.BlockSpec((HIDDEN, 2), lambda i: (0, 0)),
            pl.BlockSpec((1, 2), lambda i: (0, 0)),
        ],
        out_specs=pl.BlockSpec((B, L), lambda i: (0, 0)),
        out_shape=jax.ShapeDtypeStruct((B, L), jnp.float32),
        scratch_shapes=[pltpu.VMEM((D, NUM_CLUSTERS), jnp.float32)],
    )(flat, u, attention_mask, centroids, W_proj, b_proj.reshape(1, HIDDEN),
      W_out, b_out.reshape(1, 2))
    return out


# speedup vs baseline: 1.6795x; 1.0026x over previous
import functools

import jax
import jax.numpy as jnp
from jax.experimental import pallas as pl
from jax.experimental.pallas import tpu as pltpu

D_MODEL = 512
HIDDEN = 256
NUM_CLUSTERS = 1024
EPS = 1e-6
U_MIN = 1e-4

_PREC = jax.lax.Precision.DEFAULT


def _fused_kernel(x_ref, u_ref, m_ref, c_ref, wp_ref, bp_ref, wo_ref, bo_ref,
                  out_ref, ct_ref, h2_ref):
    i = pl.program_id(0)

    @pl.when(i == 0)
    def _():
        ct = jnp.transpose(c_ref[...])
        ct_ref[...] = ct
        h2_ref[...] = 0.5 * jnp.sum(ct * ct, axis=0, keepdims=True)

    x = x_ref[...]

    h = jax.lax.dot_general(x, wp_ref[...], (((1,), (0,)), ((), ())),
                            preferred_element_type=jnp.float32,
                            precision=_PREC)
    h = jax.nn.gelu(h + bp_ref[...][None, :])
    ab = jax.lax.dot_general(h, wo_ref[...], (((1,), (0,)), ((), ())),
                             preferred_element_type=jnp.float32,
                             precision=_PREC)
    ab = ab + bo_ref[...][None, :]
    alpha = jnp.clip(jax.nn.softplus(ab[:, 0:1]) + 1.0, 1.0, 10.0)
    beta = jnp.clip(jax.nn.softplus(ab[:, 1:2]) + 1.0, 1.0, 10.0)

    ucol = jnp.transpose(u_ref[pl.ds(i, 1), :])
    uc = jnp.clip(ucol, U_MIN, 1.0 - U_MIN)
    t = jnp.exp(jnp.log1p(-uc) / (beta + EPS))
    one_minus_t = jnp.clip(1.0 - t, EPS, 1.0)
    g = jnp.exp(jnp.log(one_minus_t) / (alpha + EPS))
    gates = jnp.clip(g, EPS, 1.0 - EPS)

    s = jax.lax.dot_general(x, ct_ref[...], (((1,), (0,)), ((), ())),
                            preferred_element_type=jnp.float32,
                            precision=_PREC)
    e = s - h2_ref[...]
    emax = jnp.max(e, axis=1, keepdims=True)
    entity = (e[:, 0:1] >= emax).astype(jnp.float32)

    res = jnp.transpose(gates * entity)
    mrow = m_ref[pl.ds(i, 1), :]
    out_ref[pl.ds(i, 1), :] = res * mrow * mrow


@functools.partial(jax.jit, static_argnames=())
def kernel(embeddings, attention_mask, centroids, u, W_proj, b_proj, W_out,
           b_out):
    B, L, D = embeddings.shape
    N = B * L
    R = L
    flat = embeddings.reshape(N, D)

    out = pl.pallas_call(
        _fused_kernel,
        grid=(B,),
        in_specs=[
            pl.BlockSpec((R, D), lambda i: (i, 0)),
            pl.BlockSpec((B, L), lambda i: (0, 0)),
            pl.BlockSpec((B, L), lambda i: (0, 0)),
            pl.BlockSpec((NUM_CLUSTERS, D), lambda i: (0, 0)),
            pl.BlockSpec((D, HIDDEN), lambda i: (0, 0)),
            pl.BlockSpec((HIDDEN,), lambda i: (0,)),
            pl.BlockSpec((HIDDEN, 2), lambda i: (0, 0)),
            pl.BlockSpec((2,), lambda i: (0,)),
        ],
        out_specs=pl.BlockSpec((B, L), lambda i: (0, 0)),
        out_shape=jax.ShapeDtypeStruct((B, L), jnp.float32),
        scratch_shapes=[pltpu.VMEM((D, NUM_CLUSTERS), jnp.float32),
                        pltpu.VMEM((1, NUM_CLUSTERS), jnp.float32)],
    )(flat, u, attention_mask, centroids, W_proj, b_proj, W_out, b_out)
    return out
